# baseline (speedup 1.0000x reference)
import functools

import jax
import jax.numpy as jnp
from jax import lax
from jax.experimental import pallas as pl
from jax.experimental.pallas import tpu as pltpu
from jax.experimental.pallas import tpu_sc as plsc

N_NODES = 10000
N_PAD = 10240
D_FEAT = 128
N_EDGES = 320000
BLK = 128
E_PAD = 327680
NBLK = E_PAD // BLK
NCORE = 2
NSUB = 16
NB = NBLK // (NCORE * NSUB)
ZCHUNK = 128

_mesh = plsc.VectorSubcoreMesh(core_axis_name="c", subcore_axis_name="s")


@functools.partial(
    pl.kernel,
    mesh=_mesh,
    out_type=jax.ShapeDtypeStruct((NCORE, N_PAD, D_FEAT), jnp.float32),
    scratch_types=[
        pltpu.VMEM((BLK,), jnp.int32),
        pltpu.VMEM((BLK,), jnp.int32),
        pltpu.VMEM((BLK,), jnp.int32),
        pltpu.VMEM((BLK,), jnp.int32),
        pltpu.VMEM((BLK, D_FEAT), jnp.float32),
        pltpu.VMEM((BLK, D_FEAT), jnp.float32),
        pltpu.VMEM_SHARED((N_PAD, D_FEAT), jnp.float32),
        pltpu.SemaphoreType.DMA,
        pltpu.SemaphoreType.DMA,
        pltpu.SemaphoreType.DMA,
        pltpu.SemaphoreType.DMA,
    ],
)
def _sc_agg(x_hbm, src_hbm, dst_hbm, out_hbm,
            srcb0, srcb1, dstb0, dstb1, rows0, rows1, acc_sh,
            isem0, isem1, gsem0, gsem1):
    c = lax.axis_index("c")
    s = lax.axis_index("s")
    srcb = (srcb0, srcb1)
    dstb = (dstb0, dstb1)
    rows = (rows0, rows1)
    isem = (isem0, isem1)
    gsem = (gsem0, gsem1)
    rpt = N_PAD // NSUB
    start = (c * (NBLK // NCORE) + s * NB) * BLK

    def zrow(i, carry):
        for j in range(D_FEAT // 16):
            rows0[i, pl.ds(j * 16, 16)] = jnp.zeros((16,), jnp.float32)
        return carry

    lax.fori_loop(0, ZCHUNK, zrow, 0)
    for k in range(rpt // ZCHUNK):
        pltpu.sync_copy(
            rows0, acc_sh.at[pl.ds(s * rpt + k * ZCHUNK, ZCHUNK)])
    plsc.subcore_barrier()

    def blk(j, carry):
        base = start + j * BLK
        pltpu.sync_copy(src_hbm.at[pl.ds(base, BLK)], srcb0)
        pltpu.sync_copy(dst_hbm.at[pl.ds(base, BLK)], dstb0)
        pltpu.async_copy(x_hbm.at[srcb0], rows0, gsem0).wait()
        pltpu.sync_copy(rows0, acc_sh.at[dstb0], add=True)
        return carry

    lax.fori_loop(0, NB, blk, 0)

    plsc.subcore_barrier()
    pltpu.sync_copy(
        acc_sh.at[pl.ds(s * rpt, rpt)],
        out_hbm.at[c, pl.ds(s * rpt, rpt)])


def _add_body(a_ref, b_ref, o_ref):
    o_ref[...] = a_ref[...] + b_ref[...]


_tc_add = pl.pallas_call(
    _add_body,
    out_shape=jax.ShapeDtypeStruct((N_PAD, D_FEAT), jnp.float32),
    grid=(10,),
    in_specs=[
        pl.BlockSpec((N_PAD // 10, D_FEAT), lambda i: (i, 0)),
        pl.BlockSpec((N_PAD // 10, D_FEAT), lambda i: (i, 0)),
    ],
    out_specs=pl.BlockSpec((N_PAD // 10, D_FEAT), lambda i: (i, 0)),
)


def kernel(x, edge_index):
    src = edge_index[0].astype(jnp.int32)
    dst = edge_index[1].astype(jnp.int32)
    pad = E_PAD - N_EDGES
    src1 = jnp.pad(src, (0, pad), constant_values=N_NODES)
    dst1 = jnp.pad(dst, (0, pad), constant_values=N_NODES)
    xp = jnp.pad(x, ((0, N_PAD - N_NODES), (0, 0)))
    parts = _sc_agg(xp, src1, dst1)
    out = _tc_add(parts[0], parts[1])
    return out[:N_NODES]


# speedup vs baseline: 1.0001x; 1.0001x over previous
import functools

import jax
import jax.numpy as jnp
from jax import lax
from jax.experimental import pallas as pl
from jax.experimental.pallas import tpu as pltpu
from jax.experimental.pallas import tpu_sc as plsc

N_NODES = 10000
N_PAD = 10240
D_FEAT = 128
N_EDGES = 320000
BLK = 128
E_PAD = 327680
NBLK = E_PAD // BLK
NCORE = 2
NSUB = 16
NB = NBLK // (NCORE * NSUB)
ZCHUNK = 128

_mesh = plsc.VectorSubcoreMesh(core_axis_name="c", subcore_axis_name="s")


@functools.partial(
    pl.kernel,
    mesh=_mesh,
    out_type=jax.ShapeDtypeStruct((NCORE, N_PAD, D_FEAT), jnp.float32),
    scratch_types=[
        pltpu.VMEM((BLK,), jnp.int32),
        pltpu.VMEM((BLK,), jnp.int32),
        pltpu.VMEM((BLK,), jnp.int32),
        pltpu.VMEM((BLK,), jnp.int32),
        pltpu.VMEM((BLK, D_FEAT), jnp.float32),
        pltpu.VMEM((BLK, D_FEAT), jnp.float32),
        pltpu.VMEM_SHARED((N_PAD, D_FEAT), jnp.float32),
        pltpu.SemaphoreType.DMA,
        pltpu.SemaphoreType.DMA,
        pltpu.SemaphoreType.DMA,
        pltpu.SemaphoreType.DMA,
    ],
)
def _sc_agg(x_hbm, src_hbm, dst_hbm, out_hbm,
            srcb0, srcb1, dstb0, dstb1, rows0, rows1, acc_sh,
            isem0, isem1, gsem0, gsem1):
    c = lax.axis_index("c")
    s = lax.axis_index("s")
    srcb = (srcb0, srcb1)
    dstb = (dstb0, dstb1)
    rows = (rows0, rows1)
    isem = (isem0, isem1)
    gsem = (gsem0, gsem1)
    rpt = N_PAD // NSUB
    start = (c * (NBLK // NCORE) + s * NB) * BLK

    def zrow(i, carry):
        for j in range(D_FEAT // 16):
            rows0[i, pl.ds(j * 16, 16)] = jnp.zeros((16,), jnp.float32)
        return carry

    lax.fori_loop(0, ZCHUNK, zrow, 0)
    for k in range(rpt // ZCHUNK):
        pltpu.sync_copy(
            rows0, acc_sh.at[pl.ds(s * rpt + k * ZCHUNK, ZCHUNK)])
    plsc.subcore_barrier()

    def blk(j, carry):
        base = start + j * BLK
        pltpu.sync_copy(src_hbm.at[pl.ds(base, BLK)], srcb0)
        pltpu.sync_copy(dst_hbm.at[pl.ds(base, BLK)], dstb0)
        pltpu.async_copy(x_hbm.at[srcb0], rows0, gsem0).wait()
        pltpu.sync_copy(rows0, acc_sh.at[dstb0], add=True)
        return carry

    nb = jnp.where(s >= 0, NB, 0)
    lax.fori_loop(0, nb, blk, 0)

    plsc.subcore_barrier()
    pltpu.sync_copy(
        acc_sh.at[pl.ds(s * rpt, rpt)],
        out_hbm.at[c, pl.ds(s * rpt, rpt)])


def _add_body(a_ref, b_ref, o_ref):
    o_ref[...] = a_ref[...] + b_ref[...]


_tc_add = pl.pallas_call(
    _add_body,
    out_shape=jax.ShapeDtypeStruct((N_PAD, D_FEAT), jnp.float32),
    grid=(10,),
    in_specs=[
        pl.BlockSpec((N_PAD // 10, D_FEAT), lambda i: (i, 0)),
        pl.BlockSpec((N_PAD // 10, D_FEAT), lambda i: (i, 0)),
    ],
    out_specs=pl.BlockSpec((N_PAD // 10, D_FEAT), lambda i: (i, 0)),
)


def kernel(x, edge_index):
    src = edge_index[0].astype(jnp.int32)
    dst = edge_index[1].astype(jnp.int32)
    pad = E_PAD - N_EDGES
    src1 = jnp.pad(src, (0, pad), constant_values=N_NODES)
    dst1 = jnp.pad(dst, (0, pad), constant_values=N_NODES)
    xp = jnp.pad(x, ((0, N_PAD - N_NODES), (0, 0)))
    parts = _sc_agg(xp, src1, dst1)
    out = _tc_add(parts[0], parts[1])
    return out[:N_NODES]


# speedup vs baseline: 2.2134x; 2.2133x over previous
import functools

import jax
import jax.numpy as jnp
from jax import lax
from jax.experimental import pallas as pl
from jax.experimental.pallas import tpu as pltpu
from jax.experimental.pallas import tpu_sc as plsc

N_NODES = 10000
N_PAD = 10240
D_FEAT = 128
N_EDGES = 320000
BLK = 128
E_PAD = 327680
NBLK = E_PAD // BLK
NCORE = 2
NSUB = 16
NB = NBLK // (NCORE * NSUB)
ZCHUNK = 128

_mesh = plsc.VectorSubcoreMesh(core_axis_name="c", subcore_axis_name="s")


@functools.partial(
    pl.kernel,
    mesh=_mesh,
    out_type=jax.ShapeDtypeStruct((NCORE, N_PAD, D_FEAT), jnp.float32),
    scratch_types=[
        pltpu.VMEM((BLK,), jnp.int32),
        pltpu.VMEM((BLK,), jnp.int32),
        pltpu.VMEM((BLK,), jnp.int32),
        pltpu.VMEM((BLK,), jnp.int32),
        pltpu.VMEM((BLK, D_FEAT), jnp.float32),
        pltpu.VMEM((BLK, D_FEAT), jnp.float32),
        pltpu.VMEM_SHARED((N_PAD, D_FEAT), jnp.float32),
        pltpu.SemaphoreType.DMA,
        pltpu.SemaphoreType.DMA,
        pltpu.SemaphoreType.DMA,
        pltpu.SemaphoreType.DMA,
    ],
)
def _sc_agg(x_hbm, src_hbm, dst_hbm, out_hbm,
            srcb0, srcb1, dstb0, dstb1, rows0, rows1, acc_sh,
            isem0, isem1, gsem0, gsem1):
    c = lax.axis_index("c")
    s = lax.axis_index("s")
    srcb = (srcb0, srcb1)
    dstb = (dstb0, dstb1)
    rows = (rows0, rows1)
    isem = (isem0, isem1)
    gsem = (gsem0, gsem1)
    rpt = N_PAD // NSUB
    start = (c * (NBLK // NCORE) + s * NB) * BLK

    def zrow(i, carry):
        for j in range(D_FEAT // 16):
            rows0[i, pl.ds(j * 16, 16)] = jnp.zeros((16,), jnp.float32)
        return carry

    lax.fori_loop(0, ZCHUNK, zrow, 0)
    for k in range(rpt // ZCHUNK):
        pltpu.sync_copy(
            rows0, acc_sh.at[pl.ds(s * rpt + k * ZCHUNK, ZCHUNK)])
    plsc.subcore_barrier()

    def blk(j, carry):
        base = start + j * BLK
        pltpu.sync_copy(src_hbm.at[pl.ds(base, BLK)], srcb0)
        pltpu.sync_copy(dst_hbm.at[pl.ds(base, BLK)], dstb0)
        pltpu.async_copy(x_hbm.at[srcb0], rows0, gsem0).wait()
        pltpu.sync_copy(rows0, acc_sh.at[dstb0], add=True)
        return carry

    nb = jnp.where(s >= 0, NB, 0)
    lax.fori_loop(0, nb, blk, 0)

    plsc.subcore_barrier()
    pltpu.sync_copy(
        acc_sh.at[pl.ds(s * rpt, rpt)],
        out_hbm.at[c, pl.ds(s * rpt, rpt)])


def _add_body(a_ref, b_ref, o_ref):
    o_ref[...] = a_ref[...] + b_ref[...]


_tc_add = pl.pallas_call(
    _add_body,
    out_shape=jax.ShapeDtypeStruct((N_PAD, D_FEAT), jnp.float32),
    grid=(10,),
    in_specs=[
        pl.BlockSpec((N_PAD // 10, D_FEAT), lambda i: (i, 0)),
        pl.BlockSpec((N_PAD // 10, D_FEAT), lambda i: (i, 0)),
    ],
    out_specs=pl.BlockSpec((N_PAD // 10, D_FEAT), lambda i: (i, 0)),
)


def kernel(x, edge_index):
    src = edge_index[0].astype(jnp.int32)
    dst = edge_index[1].astype(jnp.int32)
    pad = E_PAD - N_EDGES
    pad_idx = N_NODES + (jnp.arange(pad, dtype=jnp.int32) % (N_PAD - N_NODES))
    src1 = jnp.concatenate([src, pad_idx])
    dst1 = jnp.concatenate([dst, pad_idx])
    xp = jnp.pad(x, ((0, N_PAD - N_NODES), (0, 0)))
    parts = _sc_agg(xp, src1, dst1)
    out = _tc_add(parts[0], parts[1])
    return out[:N_NODES]


# speedup vs baseline: 4.0998x; 1.8523x over previous
import functools

import jax
import jax.numpy as jnp
from jax import lax
from jax.experimental import pallas as pl
from jax.experimental.pallas import tpu as pltpu
from jax.experimental.pallas import tpu_sc as plsc

N_NODES = 10000
N_PAD = 10240
D_FEAT = 128
N_EDGES = 320000
BLK = 128
E_PAD = 327680
NBLK = E_PAD // BLK
NCORE = 2
NSUB = 16
NB = NBLK // (NCORE * NSUB)
ZCHUNK = 128

_mesh = plsc.VectorSubcoreMesh(core_axis_name="c", subcore_axis_name="s")


@functools.partial(
    pl.kernel,
    mesh=_mesh,
    out_type=jax.ShapeDtypeStruct((NCORE, N_PAD, D_FEAT), jnp.float32),
    scratch_types=[
        pltpu.VMEM((NB // 2, BLK), jnp.int32),
        pltpu.VMEM((NB // 2, BLK), jnp.int32),
        pltpu.VMEM((BLK, D_FEAT), jnp.float32),
        pltpu.VMEM((BLK, D_FEAT), jnp.float32),
        pltpu.VMEM_SHARED((N_PAD, D_FEAT), jnp.float32),
        pltpu.SemaphoreType.DMA,
        pltpu.SemaphoreType.DMA,
    ],
)
def _sc_agg(x_hbm, src_hbm, dst_hbm, out_hbm,
            srcb, dstb, rows0, rows1, acc_sh, gsem0, gsem1):
    c = lax.axis_index("c")
    s = lax.axis_index("s")
    rows = (rows0, rows1)
    gsem = (gsem0, gsem1)
    rpt = N_PAD // NSUB
    start_blk = c * (NBLK // NCORE) + s * NB
    half = NB // 2

    def zrow(i, carry):
        for j in range(D_FEAT // 16):
            rows0[i, pl.ds(j * 16, 16)] = jnp.zeros((16,), jnp.float32)
        return carry

    lax.fori_loop(0, ZCHUNK, zrow, 0)
    for k in range(rpt // ZCHUNK):
        pltpu.sync_copy(
            rows0, acc_sh.at[pl.ds(s * rpt + k * ZCHUNK, ZCHUNK)])
    plsc.subcore_barrier()

    for p in range(2):
        pltpu.sync_copy(src_hbm.at[pl.ds(start_blk + p * half, half)], srcb)
        pltpu.sync_copy(dst_hbm.at[pl.ds(start_blk + p * half, half)], dstb)
        pltpu.async_copy(x_hbm.at[srcb.at[0]], rows0, gsem0)
        pltpu.async_copy(x_hbm.at[srcb.at[1]], rows1, gsem1)

        def pair(g, carry):
            for b in range(2):
                j = 2 * g + b
                pltpu.make_async_copy(
                    x_hbm.at[srcb.at[j]], rows[b], gsem[b]).wait()
                pltpu.sync_copy(rows[b], acc_sh.at[dstb.at[j]], add=True)
                pltpu.async_copy(x_hbm.at[srcb.at[j + 2]], rows[b], gsem[b])
            return carry

        lax.fori_loop(0, half // 2 - 1, pair, 0)
        for b in range(2):
            j = half - 2 + b
            pltpu.make_async_copy(
                x_hbm.at[srcb.at[j]], rows[b], gsem[b]).wait()
            pltpu.sync_copy(rows[b], acc_sh.at[dstb.at[j]], add=True)

    plsc.subcore_barrier()
    pltpu.sync_copy(
        acc_sh.at[pl.ds(s * rpt, rpt)],
        out_hbm.at[c, pl.ds(s * rpt, rpt)])


def _add_body(a_ref, b_ref, o_ref):
    o_ref[...] = a_ref[...] + b_ref[...]


_tc_add = pl.pallas_call(
    _add_body,
    out_shape=jax.ShapeDtypeStruct((N_PAD, D_FEAT), jnp.float32),
    grid=(10,),
    in_specs=[
        pl.BlockSpec((N_PAD // 10, D_FEAT), lambda i: (i, 0)),
        pl.BlockSpec((N_PAD // 10, D_FEAT), lambda i: (i, 0)),
    ],
    out_specs=pl.BlockSpec((N_PAD // 10, D_FEAT), lambda i: (i, 0)),
)


def kernel(x, edge_index):
    src = edge_index[0].astype(jnp.int32)
    dst = edge_index[1].astype(jnp.int32)
    pad = E_PAD - N_EDGES
    pad_idx = N_NODES + (jnp.arange(pad, dtype=jnp.int32) % (N_PAD - N_NODES))
    src1 = jnp.concatenate([src, pad_idx]).reshape(NBLK, BLK)
    dst1 = jnp.concatenate([dst, pad_idx]).reshape(NBLK, BLK)
    xp = jnp.pad(x, ((0, N_PAD - N_NODES), (0, 0)))
    parts = _sc_agg(xp, src1, dst1)
    out = _tc_add(parts[0], parts[1])
    return out[:N_NODES]
